# baseline (speedup 1.0000x reference)
import functools

import jax
import jax.numpy as jnp
from jax.experimental import pallas as pl
from jax.experimental.pallas import tpu as pltpu

_E = 8
_K = 2
_H = 768
_TBLK = 2048


def _router_block(hs_ref, w_ref, b_ref, scores_ref, idx_ref):
    hs = hs_ref[...]
    w = w_ref[...]
    logits = jax.lax.dot_general(
        hs, w, (((1,), (1,)), ((), ())), preferred_element_type=jnp.float32)
    logits = logits + b_ref[...]

    e_iota = jax.lax.broadcasted_iota(jnp.int32, logits.shape, 1)
    m1 = jnp.max(logits, axis=1, keepdims=True)
    i1 = jnp.min(jnp.where(logits == m1, e_iota, _E), axis=1, keepdims=True)
    masked = jnp.where(e_iota == i1, -jnp.inf, logits)
    m2 = jnp.max(masked, axis=1, keepdims=True)
    i2 = jnp.min(jnp.where(masked == m2, e_iota, _E), axis=1, keepdims=True)

    s = jnp.exp(m2 - m1)
    r = 1.0 / (1.0 + s)
    p1 = r
    p2 = s * r
    scores_ref[...] = (jnp.where(e_iota == i1, p1, 0.0)
                       + jnp.where(e_iota == i2, p2, 0.0))
    idx_ref[...] = jnp.concatenate([i1, i2], axis=1)


@jax.jit
def kernel(hidden_states, router_weight, router_bias):
    t = hidden_states.shape[0]
    grid = (t // _TBLK,)
    scores, idx = pl.pallas_call(
        _router_block,
        grid=grid,
        in_specs=[
            pl.BlockSpec((_TBLK, _H), lambda i: (i, 0)),
            pl.BlockSpec((_E, _H), lambda i: (0, 0)),
            pl.BlockSpec((1, _E), lambda i: (0, 0)),
        ],
        out_specs=[
            pl.BlockSpec((_TBLK, _E), lambda i: (i, 0)),
            pl.BlockSpec((_TBLK, _K), lambda i: (i, 0)),
        ],
        out_shape=[
            jax.ShapeDtypeStruct((t, _E), jnp.float32),
            jax.ShapeDtypeStruct((t, _K), jnp.int32),
        ],
    )(hidden_states, router_weight, router_bias.reshape(1, _E))
    return scores, idx


# speedup vs baseline: 1.0887x; 1.0887x over previous
import jax
import jax.numpy as jnp
from jax.experimental import pallas as pl

_E = 8
_K = 2
_H = 768
_TBLK = 2048


def _router_block(w_ref, b_ref, hs_ref, scores_ref, idx_ref):
    hs = hs_ref[...]
    w = w_ref[...]
    logits = jax.lax.dot_general(
        w, hs, (((1,), (1,)), ((), ())), preferred_element_type=jnp.float32)
    logits = logits + b_ref[...]

    e_iota = jax.lax.broadcasted_iota(jnp.int32, logits.shape, 0)
    m1 = jnp.max(logits, axis=0, keepdims=True)
    i1 = jnp.min(jnp.where(logits == m1, e_iota, _E), axis=0, keepdims=True)
    masked = jnp.where(e_iota == i1, -jnp.inf, logits)
    m2 = jnp.max(masked, axis=0, keepdims=True)
    i2 = jnp.min(jnp.where(masked == m2, e_iota, _E), axis=0, keepdims=True)

    s = jnp.exp(m2 - m1)
    r = 1.0 / (1.0 + s)
    scores_t = (jnp.where(e_iota == i1, r, 0.0)
                + jnp.where(e_iota == i2, s * r, 0.0))
    scores_ref[...] = scores_t.T
    idx_ref[...] = jnp.concatenate([i1, i2], axis=0).T


@jax.jit
def kernel(hidden_states, router_weight, router_bias):
    t = hidden_states.shape[0]
    grid = (t // _TBLK,)
    scores, idx = pl.pallas_call(
        _router_block,
        grid=grid,
        in_specs=[
            pl.BlockSpec((_E, _H), lambda i: (0, 0)),
            pl.BlockSpec((_E, 1), lambda i: (0, 0)),
            pl.BlockSpec((_TBLK, _H), lambda i: (i, 0)),
        ],
        out_specs=[
            pl.BlockSpec((_TBLK, _E), lambda i: (i, 0)),
            pl.BlockSpec((_TBLK, _K), lambda i: (i, 0)),
        ],
        out_shape=[
            jax.ShapeDtypeStruct((t, _E), jnp.float32),
            jax.ShapeDtypeStruct((t, _K), jnp.int32),
        ],
    )(router_weight, router_bias.reshape(_E, 1), hidden_states)
    return scores, idx


# speedup vs baseline: 1.0936x; 1.0045x over previous
import jax
import jax.numpy as jnp
from jax.experimental import pallas as pl
from jax.experimental.pallas import tpu as pltpu

_E = 8
_K = 2
_H = 768
_TBLK = 2048


def _router_block(w_ref, b_ref, hs_ref, scores_ref, idx_ref):
    hs = hs_ref[...]
    w = w_ref[...]
    logits = jax.lax.dot_general(
        w, hs, (((1,), (1,)), ((), ())), preferred_element_type=jnp.float32)
    logits = logits + b_ref[...]

    e_iota = jax.lax.broadcasted_iota(jnp.int32, logits.shape, 0)
    m1 = jnp.max(logits, axis=0, keepdims=True)
    i1 = jnp.min(jnp.where(logits == m1, e_iota, _E), axis=0, keepdims=True)
    masked = jnp.where(e_iota == i1, -jnp.inf, logits)
    m2 = jnp.max(masked, axis=0, keepdims=True)
    i2 = jnp.min(jnp.where(masked == m2, e_iota, _E), axis=0, keepdims=True)

    s = jnp.exp(m2 - m1)
    r = 1.0 / (1.0 + s)
    scores_t = (jnp.where(e_iota == i1, r, 0.0)
                + jnp.where(e_iota == i2, s * r, 0.0))
    scores_ref[...] = scores_t.T
    idx_ref[...] = jnp.concatenate([i1, i2], axis=0).T


@jax.jit
def kernel(hidden_states, router_weight, router_bias):
    t = hidden_states.shape[0]
    grid = (t // _TBLK,)
    scores, idx = pl.pallas_call(
        _router_block,
        grid=grid,
        in_specs=[
            pl.BlockSpec((_E, _H), lambda i: (0, 0)),
            pl.BlockSpec((_E, 1), lambda i: (0, 0)),
            pl.BlockSpec((_TBLK, _H), lambda i: (i, 0)),
        ],
        out_specs=[
            pl.BlockSpec((_TBLK, _E), lambda i: (i, 0)),
            pl.BlockSpec((_TBLK, _K), lambda i: (i, 0)),
        ],
        out_shape=[
            jax.ShapeDtypeStruct((t, _E), jnp.float32),
            jax.ShapeDtypeStruct((t, _K), jnp.int32),
        ],
        compiler_params=pltpu.CompilerParams(
            dimension_semantics=("parallel",)),
    )(router_weight, router_bias.reshape(_E, 1), hidden_states)
    return scores, idx


# speedup vs baseline: 1.9886x; 1.8185x over previous
import jax
import jax.numpy as jnp
from jax.experimental import pallas as pl
from jax.experimental.pallas import tpu as pltpu

_E = 8
_K = 2
_H = 768
_TBLK = 2048


def _router_block(w_ref, b_ref, hs_ref, scores_ref, idx_ref):
    hs = hs_ref[...]
    w = w_ref[...]
    logits = jax.lax.dot_general(
        w, hs, (((1,), (1,)), ((), ())), preferred_element_type=jnp.float32)
    logits = logits + b_ref[...]

    e_iota = jax.lax.broadcasted_iota(jnp.int32, logits.shape, 0)
    m1 = jnp.max(logits, axis=0, keepdims=True)
    i1 = jnp.min(jnp.where(logits == m1, e_iota, _E), axis=0, keepdims=True)
    masked = jnp.where(e_iota == i1, -jnp.inf, logits)
    m2 = jnp.max(masked, axis=0, keepdims=True)
    i2 = jnp.min(jnp.where(masked == m2, e_iota, _E), axis=0, keepdims=True)

    s = jnp.exp(m2 - m1)
    r = 1.0 / (1.0 + s)
    scores_t = (jnp.where(e_iota == i1, r, 0.0)
                + jnp.where(e_iota == i2, s * r, 0.0))
    scores_ref[...] = scores_t
    idx_ref[...] = jnp.concatenate([i1, i2], axis=0)


@jax.jit
def kernel(hidden_states, router_weight, router_bias):
    t = hidden_states.shape[0]
    grid = (t // _TBLK,)
    scores_p, idx_p = pl.pallas_call(
        _router_block,
        grid=grid,
        in_specs=[
            pl.BlockSpec((_E, _H), lambda i: (0, 0)),
            pl.BlockSpec((_E, 1), lambda i: (0, 0)),
            pl.BlockSpec((_TBLK, _H), lambda i: (i, 0)),
        ],
        out_specs=[
            pl.BlockSpec((_E, _TBLK), lambda i: (0, i)),
            pl.BlockSpec((_K, _TBLK), lambda i: (0, i)),
        ],
        out_shape=[
            jax.ShapeDtypeStruct((_E, t), jnp.float32),
            jax.ShapeDtypeStruct((_K, t), jnp.int32),
        ],
        compiler_params=pltpu.CompilerParams(
            dimension_semantics=("parallel",)),
    )(router_weight, router_bias.reshape(_E, 1), hidden_states)
    return scores_p.T, idx_p.T


# speedup vs baseline: 2.0298x; 1.0207x over previous
import jax
import jax.numpy as jnp
from jax.experimental import pallas as pl
from jax.experimental.pallas import tpu as pltpu

_E = 8
_K = 2
_H = 768
_TBLK = 4096


def _router_block(w_ref, b_ref, hs_ref, scores_ref, idx_ref):
    hs = hs_ref[...]
    w = w_ref[...]
    logits = jax.lax.dot_general(
        w, hs, (((1,), (1,)), ((), ())), preferred_element_type=jnp.float32)
    logits = logits + b_ref[...]

    e_iota = jax.lax.broadcasted_iota(jnp.int32, logits.shape, 0)
    m1 = jnp.max(logits, axis=0, keepdims=True)
    i1 = jnp.min(jnp.where(logits == m1, e_iota, _E), axis=0, keepdims=True)
    masked = jnp.where(e_iota == i1, -jnp.inf, logits)
    m2 = jnp.max(masked, axis=0, keepdims=True)
    i2 = jnp.min(jnp.where(masked == m2, e_iota, _E), axis=0, keepdims=True)

    s = jnp.exp(m2 - m1)
    r = 1.0 / (1.0 + s)
    scores_t = (jnp.where(e_iota == i1, r, 0.0)
                + jnp.where(e_iota == i2, s * r, 0.0))
    scores_ref[...] = scores_t
    idx_ref[...] = jnp.concatenate([i1, i2], axis=0)


@jax.jit
def kernel(hidden_states, router_weight, router_bias):
    t = hidden_states.shape[0]
    grid = (t // _TBLK,)
    scores_p, idx_p = pl.pallas_call(
        _router_block,
        grid=grid,
        in_specs=[
            pl.BlockSpec((_E, _H), lambda i: (0, 0)),
            pl.BlockSpec((_E, 1), lambda i: (0, 0)),
            pl.BlockSpec((_TBLK, _H), lambda i: (i, 0)),
        ],
        out_specs=[
            pl.BlockSpec((_E, _TBLK), lambda i: (0, i)),
            pl.BlockSpec((_K, _TBLK), lambda i: (0, i)),
        ],
        out_shape=[
            jax.ShapeDtypeStruct((_E, t), jnp.float32),
            jax.ShapeDtypeStruct((_K, t), jnp.int32),
        ],
        compiler_params=pltpu.CompilerParams(
            dimension_semantics=("parallel",)),
    )(router_weight, router_bias.reshape(_E, 1), hidden_states)
    return scores_p.T, idx_p.T
